# Initial kernel scaffold; baseline (speedup 1.0000x reference)
import jax
import jax.numpy as jnp
from jax.experimental import pallas as pl


def kernel(x, y, bias):
    raise NotImplementedError("write your pallas kernel here")


# speedup vs baseline: 509.9470x; 509.9470x over previous
import functools

import jax
import jax.numpy as jnp
from jax.experimental import pallas as pl
from jax.experimental.pallas import tpu as pltpu

DOMAIN_MIN = -2.0
DOMAIN_MAX = 2.0


def _kan_kernel(x_ref, yt_ref, bias_ref, out_ref, a_ref, *, K, d_in, d_out, h):
    @pl.when(pl.program_id(0) == 0)
    def _build_table():
        yt = yt_ref[...]
        delta = (yt[1:] - yt[:-1]) * (1.0 / h)
        d0 = delta[:-1]
        d1 = delta[1:]
        same = (d0 * d1) > 0
        denom = d0 + d1
        small = jnp.abs(denom) < 1e-12
        safe_denom = jnp.where(small, jnp.ones_like(denom), denom)
        hm = 2.0 * d0 * d1 / safe_denom
        hm = jnp.where(small, jnp.zeros_like(hm), hm)
        mid = jnp.where(same, hm, jnp.zeros_like(hm))
        m0 = (3.0 * delta[0] - delta[1]) / 2.0
        mN = (3.0 * delta[-1] - delta[-2]) / 2.0
        m0 = jnp.where(m0 * delta[0] <= 0, jnp.zeros_like(m0), m0)
        mN = jnp.where(mN * delta[-1] <= 0, jnp.zeros_like(mN), mN)
        cond0 = (delta[0] * delta[1] < 0) & (jnp.abs(m0) > jnp.abs(3.0 * delta[0]))
        m0 = jnp.where(cond0, 3.0 * delta[0], m0)
        condN = (delta[-1] * delta[-2] < 0) & (jnp.abs(mN) > jnp.abs(3.0 * delta[-1]))
        mN = jnp.where(condN, 3.0 * delta[-1], mN)

        a_ref[0:d_in, :] = yt[0]
        a_ref[(K - 1) * d_in:K * d_in, :] = yt[K - 1]
        a_ref[K * d_in:(K + 1) * d_in, :] = h * m0
        a_ref[(2 * K - 1) * d_in:2 * K * d_in, :] = h * mN
        for k in range(1, K - 1):
            a_ref[k * d_in:(k + 1) * d_in, :] = yt[k]
            a_ref[(K + k) * d_in:(K + k + 1) * d_in, :] = h * mid[k - 1]

    x = x_ref[...]
    B_blk = x.shape[0]
    xc = jnp.clip(x, DOMAIN_MIN, DOMAIN_MAX)
    t = (xc - DOMAIN_MIN) * (1.0 / h)
    idx = jnp.clip(jnp.floor(t).astype(jnp.int32), 0, K - 2)
    u = t - idx.astype(x.dtype)
    u2 = u * u
    u3 = u2 * u
    h00 = 2.0 * u3 - 3.0 * u2 + 1.0
    h10 = u3 - 2.0 * u2 + u
    h01 = -2.0 * u3 + 3.0 * u2
    h11 = u3 - u2

    idx_t = jnp.concatenate([idx] * K, axis=1)
    kk = jax.lax.broadcasted_iota(jnp.int32, (B_blk, K * d_in), 1) // d_in
    at0 = idx_t == kk
    at1 = idx_t == (kk - 1)
    zero = jnp.zeros_like(idx_t, dtype=x.dtype)
    wy = (jnp.where(at0, jnp.concatenate([h00] * K, axis=1), zero)
          + jnp.where(at1, jnp.concatenate([h01] * K, axis=1), zero))
    wm = (jnp.where(at0, jnp.concatenate([h10] * K, axis=1), zero)
          + jnp.where(at1, jnp.concatenate([h11] * K, axis=1), zero))
    w = jnp.concatenate([wy, wm], axis=1)

    acc = jax.lax.dot_general(
        w, a_ref[...],
        dimension_numbers=(((1,), (0,)), ((), ())),
        preferred_element_type=jnp.float32,
    )
    out_ref[...] = acc + bias_ref[...][None, :]


def kernel(x, y, bias):
    B, d_in = x.shape
    d_out = y.shape[1]
    K = y.shape[2]
    h = (DOMAIN_MAX - DOMAIN_MIN) / (K - 1)
    yt = jnp.transpose(y, (2, 0, 1))

    B_blk = 512
    grid = (B // B_blk,)
    return pl.pallas_call(
        functools.partial(_kan_kernel, K=K, d_in=d_in, d_out=d_out, h=h),
        grid=grid,
        in_specs=[
            pl.BlockSpec((B_blk, d_in), lambda b: (b, 0)),
            pl.BlockSpec((K, d_in, d_out), lambda b: (0, 0, 0)),
            pl.BlockSpec((d_out,), lambda b: (0,)),
        ],
        out_specs=pl.BlockSpec((B_blk, d_out), lambda b: (b, 0)),
        out_shape=jax.ShapeDtypeStruct((B, d_out), x.dtype),
        scratch_shapes=[pltpu.VMEM((2 * K * d_in, d_out), jnp.float32)],
    )(x, yt, bias)
